# Initial kernel scaffold; baseline (speedup 1.0000x reference)
import functools

import jax
import jax.numpy as jnp
from jax import lax
from jax.experimental import pallas as pl
from jax.experimental.pallas import tpu as pltpu
from jax.experimental.pallas import tpu_sc as plsc

_MAX_IDX = 64
_L = 16


def _prefix_kernel(x_ref, p_ref):
    x = x_ref[0]
    d = x.shape[-1]
    rows = lax.broadcasted_iota(jnp.int32, (_MAX_IDX + 1, _MAX_IDX), 0)
    cols = lax.broadcasted_iota(jnp.int32, (_MAX_IDX + 1, _MAX_IDX), 1)
    tri = (cols < rows).astype(jnp.float32)
    p_ref[0] = jax.lax.dot_general(
        tri, x, (((1,), (0,)), ((), ())),
        preferred_element_type=jnp.float32,
    )


def _make_sc_extract(total_spans, d, spans_per_batch):
    info = plsc.get_sparse_core_info()
    nw = info.num_cores * info.num_subcores
    spw = total_spans // nw
    mesh = plsc.VectorSubcoreMesh(core_axis_name="c", subcore_axis_name="s")

    @functools.partial(
        pl.kernel,
        mesh=mesh,
        out_type=jax.ShapeDtypeStruct((total_spans, d), jnp.float32),
        scratch_types=[
            pltpu.VMEM((spw,), jnp.int32),
            pltpu.VMEM((spw,), jnp.int32),
            pltpu.VMEM((spw,), jnp.int32),
            pltpu.VMEM((spw,), jnp.int32),
            pltpu.VMEM((spw,), jnp.float32),
            pltpu.VMEM((spw, d), jnp.float32),
            pltpu.VMEM((spw, d), jnp.float32),
            pltpu.SemaphoreType.DMA,
            pltpu.SemaphoreType.DMA,
        ],
    )
    def extract(p_hbm, starts_hbm, ends_hbm, out_hbm,
                s_v, e_v, idx_s, idx_e, invw, rows_s, rows_e,
                sem_s, sem_e):
        wid = lax.axis_index("s") * info.num_cores + lax.axis_index("c")
        base = wid * spw
        rowoff = (base // spans_per_batch) * (_MAX_IDX + 1)

        pltpu.sync_copy(starts_hbm.at[pl.ds(base, spw)], s_v)
        pltpu.sync_copy(ends_hbm.at[pl.ds(base, spw)], e_v)

        for g in range(spw // _L):
            sl = pl.ds(g * _L, _L)
            s16 = s_v[sl]
            e16 = e_v[sl]
            idx_s[sl] = s16 + rowoff
            idx_e[sl] = e16 + (rowoff + 1)
            invw[sl] = 1.0 / (e16 - s16 + 1).astype(jnp.float32)

        cp_e = pltpu.async_copy(p_hbm.at[idx_e], rows_e, sem_e)
        cp_s = pltpu.async_copy(p_hbm.at[idx_s], rows_s, sem_s)
        cp_e.wait()
        cp_s.wait()

        def scale_row(j, carry):
            inv = plsc.load_gather(invw, [j + jnp.zeros((_L,), jnp.int32)])
            for c in range(d // _L):
                sl = pl.ds(c * _L, _L)
                rows_e[j, sl] = (rows_e[j, sl] - rows_s[j, sl]) * inv
            return carry

        lax.fori_loop(0, spw, scale_row, 0)

        pltpu.sync_copy(rows_e, out_hbm.at[pl.ds(base, spw)])

    return extract


def kernel(sequence_tensor, span_indices):
    b, _, d = sequence_tensor.shape
    n = span_indices.shape[1]

    prefix = pl.pallas_call(
        _prefix_kernel,
        grid=(b,),
        in_specs=[pl.BlockSpec((1, _MAX_IDX, d), lambda i: (i, 0, 0))],
        out_specs=pl.BlockSpec((1, _MAX_IDX + 1, d), lambda i: (i, 0, 0)),
        out_shape=jax.ShapeDtypeStruct((b, _MAX_IDX + 1, d), jnp.float32),
    )(sequence_tensor)
    p_flat = prefix.reshape(b * (_MAX_IDX + 1), d)

    starts = span_indices[..., 0].reshape(-1).astype(jnp.int32)
    ends = span_indices[..., 1].reshape(-1).astype(jnp.int32)

    extract = _make_sc_extract(b * n, d, n)
    out = extract(p_flat, starts, ends)
    return out.reshape(b, n, d)


# speedup vs baseline: 115.0841x; 115.0841x over previous
import functools

import jax
import jax.numpy as jnp
from jax import lax
from jax.experimental import pallas as pl
from jax.experimental.pallas import tpu as pltpu
from jax.experimental.pallas import tpu_sc as plsc

_MAX_IDX = 64
_L = 16


def _prefix_kernel(x_ref, p_ref):
    x = x_ref[0]
    d = x.shape[-1]
    rows = lax.broadcasted_iota(jnp.int32, (_MAX_IDX + 1, _MAX_IDX), 0)
    cols = lax.broadcasted_iota(jnp.int32, (_MAX_IDX + 1, _MAX_IDX), 1)
    tri = (cols < rows).astype(jnp.float32)
    p_ref[0] = jax.lax.dot_general(
        tri, x, (((1,), (0,)), ((), ())),
        preferred_element_type=jnp.float32,
        precision=lax.Precision.HIGHEST,
    )


def _make_sc_extract(total_spans, d, spans_per_batch):
    info = plsc.get_sparse_core_info()
    nw = info.num_cores * info.num_subcores
    spw = total_spans // nw
    mesh = plsc.VectorSubcoreMesh(core_axis_name="c", subcore_axis_name="s")

    @functools.partial(
        pl.kernel,
        mesh=mesh,
        out_type=jax.ShapeDtypeStruct((total_spans, d), jnp.float32),
        scratch_types=[
            pltpu.VMEM((spw,), jnp.int32),
            pltpu.VMEM((spw,), jnp.int32),
            pltpu.VMEM((spw,), jnp.int32),
            pltpu.VMEM((spw,), jnp.int32),
            pltpu.VMEM((spw,), jnp.int32),
            pltpu.VMEM((spw, 8 * _L), jnp.float32),
            pltpu.VMEM((spw, d), jnp.float32),
            pltpu.VMEM((spw, d), jnp.float32),
            pltpu.SemaphoreType.DMA,
            pltpu.SemaphoreType.DMA,
            pltpu.SemaphoreType.DMA,
        ],
    )
    def extract(p_hbm, starts_hbm, ends_hbm, rcp_hbm, out_hbm,
                s_v, e_v, idx_s, idx_e, idx_w, inv_rows, rows_s, rows_e,
                sem_s, sem_e, sem_w):
        wid = lax.axis_index("s") * info.num_cores + lax.axis_index("c")
        base = wid * spw
        rowoff = (base // spans_per_batch) * (_MAX_IDX + 1)

        pltpu.sync_copy(starts_hbm.at[pl.ds(base, spw)], s_v)
        pltpu.sync_copy(ends_hbm.at[pl.ds(base, spw)], e_v)

        for g in range(spw // _L):
            sl = pl.ds(g * _L, _L)
            s16 = s_v[sl]
            e16 = e_v[sl]
            idx_s[sl] = s16 + rowoff
            idx_e[sl] = e16 + (rowoff + 1)
            idx_w[sl] = e16 - s16

        cp_e = pltpu.async_copy(p_hbm.at[idx_e], rows_e, sem_e)
        cp_s = pltpu.async_copy(p_hbm.at[idx_s], rows_s, sem_s)
        cp_w = pltpu.async_copy(rcp_hbm.at[idx_w], inv_rows, sem_w)
        cp_e.wait()
        cp_s.wait()
        cp_w.wait()

        def scale_row(j, carry):
            inv = inv_rows[j, pl.ds(0, _L)]
            for c in range(d // _L):
                sl = pl.ds(c * _L, _L)
                rows_e[j, sl] = (rows_e[j, sl] - rows_s[j, sl]) * inv
            return carry

        lax.fori_loop(0, spw, scale_row, 0)

        pltpu.sync_copy(rows_e, out_hbm.at[pl.ds(base, spw)])

    return extract


def kernel(sequence_tensor, span_indices):
    b, _, d = sequence_tensor.shape
    n = span_indices.shape[1]

    prefix = pl.pallas_call(
        _prefix_kernel,
        grid=(b,),
        in_specs=[pl.BlockSpec((1, _MAX_IDX, d), lambda i: (i, 0, 0))],
        out_specs=pl.BlockSpec((1, _MAX_IDX + 1, d), lambda i: (i, 0, 0)),
        out_shape=jax.ShapeDtypeStruct((b, _MAX_IDX + 1, d), jnp.float32),
    )(sequence_tensor)
    p_flat = prefix.reshape(b * (_MAX_IDX + 1), d)

    starts = span_indices[..., 0].reshape(-1).astype(jnp.int32)
    ends = span_indices[..., 1].reshape(-1).astype(jnp.int32)

    rcp = jnp.broadcast_to(
        (1.0 / (jnp.arange(1, _MAX_IDX + 1, dtype=jnp.float32)))[:, None],
        (_MAX_IDX, 8 * _L),
    )

    extract = _make_sc_extract(b * n, d, n)
    out = extract(p_flat, starts, ends, rcp)
    return out.reshape(b, n, d)


# speedup vs baseline: 128.2855x; 1.1147x over previous
import functools

import numpy as np
import jax
import jax.numpy as jnp
from jax import lax
from jax.experimental import pallas as pl
from jax.experimental.pallas import tpu as pltpu
from jax.experimental.pallas import tpu_sc as plsc

_MAX_IDX = 64
_L = 16
_RCP_W = 8 * _L
_CHUNK = 16

_RCP_TABLE = np.broadcast_to(
    (1.0 / np.arange(1, _MAX_IDX + 1, dtype=np.float32))[:, None],
    (_MAX_IDX, _RCP_W),
).copy()


def _prefix_kernel(x_ref, p_ref):
    b = x_ref.shape[0]
    d = x_ref.shape[-1]
    x = x_ref[...].reshape(b * _MAX_IDX, d)
    pr = _MAX_IDX + 1
    rows = lax.broadcasted_iota(jnp.int32, (b * pr, b * _MAX_IDX), 0)
    cols = lax.broadcasted_iota(jnp.int32, (b * pr, b * _MAX_IDX), 1)
    rb = rows // pr
    cb = cols // _MAX_IDX
    tri = ((rb == cb) & (cols - cb * _MAX_IDX < rows - rb * pr))
    p_ref[...] = jax.lax.dot_general(
        tri.astype(jnp.float32), x, (((1,), (0,)), ((), ())),
        preferred_element_type=jnp.float32,
        precision=lax.Precision.HIGHEST,
    )


def _make_sc_extract(total_spans, d, spans_per_batch):
    info = plsc.get_sparse_core_info()
    nw = info.num_cores * info.num_subcores
    spw = total_spans // nw
    nch = spw // _CHUNK
    mesh = plsc.VectorSubcoreMesh(core_axis_name="c", subcore_axis_name="s")

    @functools.partial(
        pl.kernel,
        mesh=mesh,
        out_type=jax.ShapeDtypeStruct((total_spans, d), jnp.float32),
        scratch_types=[
            pltpu.VMEM((spw,), jnp.int32),
            pltpu.VMEM((spw,), jnp.int32),
            pltpu.VMEM((spw,), jnp.int32),
            pltpu.VMEM((spw,), jnp.int32),
            pltpu.VMEM((spw,), jnp.int32),
            pltpu.VMEM((spw, _RCP_W), jnp.float32),
            pltpu.VMEM((spw, d), jnp.float32),
            pltpu.VMEM((spw, d), jnp.float32),
            pltpu.SemaphoreType.DMA,
            [pltpu.SemaphoreType.DMA] * nch,
            [pltpu.SemaphoreType.DMA] * nch,
            [pltpu.SemaphoreType.DMA] * nch,
        ],
    )
    def extract(p_hbm, starts_hbm, ends_hbm, rcp_hbm, out_hbm,
                s_v, e_v, idx_s, idx_e, idx_w, inv_rows, rows_s, rows_e,
                sem_w, sems_s, sems_e, sems_o):
        wid = lax.axis_index("s") * info.num_cores + lax.axis_index("c")
        base = wid * spw
        rowoff = (base // spans_per_batch) * (_MAX_IDX + 1)

        pltpu.sync_copy(starts_hbm.at[pl.ds(base, spw)], s_v)
        pltpu.sync_copy(ends_hbm.at[pl.ds(base, spw)], e_v)

        for g in range(spw // _L):
            sl = pl.ds(g * _L, _L)
            s16 = s_v[sl]
            e16 = e_v[sl]
            idx_s[sl] = s16 + rowoff
            idx_e[sl] = e16 + (rowoff + 1)
            idx_w[sl] = e16 - s16

        cp_w = pltpu.async_copy(rcp_hbm.at[idx_w], inv_rows, sem_w)
        cps = []
        for k in range(nch):
            ck = pl.ds(k * _CHUNK, _CHUNK)
            cps.append((
                pltpu.async_copy(p_hbm.at[idx_e.at[ck]], rows_e.at[ck],
                                 sems_e[k]),
                pltpu.async_copy(p_hbm.at[idx_s.at[ck]], rows_s.at[ck],
                                 sems_s[k]),
            ))
        cp_w.wait()

        outs = []
        for k in range(nch):
            cp_e, cp_s = cps[k]
            cp_e.wait()
            cp_s.wait()

            def scale_row(j, carry, goff=k * _CHUNK):
                row = goff + j
                inv = inv_rows[row, pl.ds(0, _L)]
                for c in range(d // _L):
                    sl = pl.ds(c * _L, _L)
                    rows_e[row, sl] = (rows_e[row, sl] - rows_s[row, sl]) * inv
                return carry

            lax.fori_loop(0, _CHUNK, scale_row, 0)
            ck = pl.ds(k * _CHUNK, _CHUNK)
            outs.append(pltpu.async_copy(
                rows_e.at[ck], out_hbm.at[pl.ds(base + k * _CHUNK, _CHUNK)],
                sems_o[k]))

        for cp in outs:
            cp.wait()

    return extract


def kernel(sequence_tensor, span_indices):
    b, _, d = sequence_tensor.shape
    n = span_indices.shape[1]

    p_flat = pl.pallas_call(
        _prefix_kernel,
        grid=(1,),
        in_specs=[pl.BlockSpec((b, _MAX_IDX, d), lambda i: (0, 0, 0))],
        out_specs=pl.BlockSpec((b * (_MAX_IDX + 1), d), lambda i: (0, 0)),
        out_shape=jax.ShapeDtypeStruct((b * (_MAX_IDX + 1), d), jnp.float32),
    )(sequence_tensor)

    starts = span_indices[..., 0].reshape(-1).astype(jnp.int32)
    ends = span_indices[..., 1].reshape(-1).astype(jnp.int32)

    extract = _make_sc_extract(b * n, d, n)
    out = extract(p_flat, starts, ends, _RCP_TABLE)
    return out.reshape(b, n, d)
